# Initial kernel scaffold; baseline (speedup 1.0000x reference)
import jax
import jax.numpy as jnp
from jax.experimental import pallas as pl


def kernel(x_o, x_a, edge_index_o, edge_index_s, W_o1, b_o1, W_s1, b_s1, W_o2, b_o2, W_s2, b_s2, disc_W, disc_b, alpha1, alpha2):
    raise NotImplementedError("write your pallas kernel here")


# speedup vs baseline: 66.5232x; 66.5232x over previous
import jax
import jax.numpy as jnp
from jax import lax
from jax.experimental import pallas as pl

N = 489
NP = 512
FEAT = 512
H1 = 256
H2 = 128
E = 31296
EP = 32768
EC = 2048
DRUG = 271
MIC = N - DRUG
GAMMA = 0.5

_OUT_SHAPES = (
    jax.ShapeDtypeStruct((DRUG, MIC), jnp.float32),
    jax.ShapeDtypeStruct((N, 2), jnp.float32),
    jax.ShapeDtypeStruct((N, 2), jnp.float32),
    jax.ShapeDtypeStruct((N, H1 * 2 // 2), jnp.float32),
)


def _dot(a, b):
    return lax.dot_general(a, b, (((1,), (0,)), ((), ())),
                           preferred_element_type=jnp.float32)


def _dot_t(a, b):
    return lax.dot_general(a, b, (((1,), (1,)), ((), ())),
                           preferred_element_type=jnp.float32)


def _build_adj(ei_ref):
    iota_r = lax.broadcasted_iota(jnp.int32, (NP, EC), 0)
    acc = jnp.zeros((NP, NP), jnp.float32)
    for c in range(EP // EC):
        src = ei_ref[0:1, c * EC:(c + 1) * EC]
        dst = ei_ref[1:2, c * EC:(c + 1) * EC]
        ohd = (iota_r == dst).astype(jnp.bfloat16)
        ohs = (iota_r == src).astype(jnp.bfloat16)
        acc = acc + _dot_t(ohd, ohs)
    return acc


def _gip(y, m):
    mn = jnp.min(y, axis=1, keepdims=True)
    mx = jnp.max(y, axis=1, keepdims=True)
    yn = (y - mn) / (mx - mn)
    k = _dot_t(yn, yn)
    r = lax.broadcasted_iota(jnp.int32, (m, m), 0)
    c = lax.broadcasted_iota(jnp.int32, (m, m), 1)
    eye = (r == c).astype(jnp.float32)
    dcol = jnp.sum(k * eye, axis=1, keepdims=True)
    md = jnp.sum(dcol) / m
    k = k / md
    dcol = dcol / md
    drow = jnp.sum(k * eye, axis=0, keepdims=True)
    dist = dcol + drow - 2.0 * k
    kk = jnp.abs(jnp.exp(dist * (-GAMMA)))
    mp = jnp.min(jnp.where(kk > 0.0, kk, jnp.inf))
    min_v = jnp.where(mp == jnp.inf, 0.0, mp)
    kk = jnp.where(kk == 0.0, min_v, kk)
    dg_c = jnp.sqrt(jnp.sum(kk * eye, axis=1, keepdims=True))
    dg_r = jnp.sqrt(jnp.sum(kk * eye, axis=0, keepdims=True))
    return kk / (dg_c * dg_r)


def _mngnn_kernel(xo_ref, xa_ref, eio_ref, eis_ref,
                  Wo1_ref, Ws1_ref, Wo2_ref, Ws2_ref, dW_ref,
                  b1o_ref, b1s_ref, b2o_ref, b2s_ref, db_ref,
                  a1_ref, a2_ref,
                  out_ref, ros_ref, rosa_ref, x2_ref):
    r = lax.broadcasted_iota(jnp.int32, (NP, NP), 0)
    c = lax.broadcasted_iota(jnp.int32, (NP, NP), 1)
    eye = (r == c).astype(jnp.float32)

    def make_A(ei_ref):
        adj = _build_adj(ei_ref)
        deg = jnp.sum(adj, axis=1, keepdims=True) + 1.0
        dinv = 1.0 / jnp.sqrt(deg)
        dinv_r = jnp.sum(dinv * eye, axis=0, keepdims=True)
        return adj * dinv * dinv_r + eye * (dinv * dinv)

    A_o = make_A(eio_ref)
    A_s = make_A(eis_ref)

    Wo1 = Wo1_ref[:]
    Ws1 = Ws1_ref[:]
    Wo2a = Wo2_ref[0:H1, :]
    Wo2b = Wo2_ref[H1:2 * H1, :]
    Ws2a = Ws2_ref[0:H1, :]
    Ws2b = Ws2_ref[H1:2 * H1, :]
    b1o = b1o_ref[:]
    b1s = b1s_ref[:]
    b2o = b2o_ref[:]
    b2s = b2s_ref[:]

    def gnn(x):
        x1o = jnp.maximum(_dot(A_o, _dot(x, Wo1)) + b1o, 0.0)
        x1s = jnp.maximum(_dot(A_s, _dot(x, Ws1)) + b1s, 0.0)
        x2o = _dot(A_o, _dot(x1o, Wo2a) + _dot(x1s, Wo2b)) + b2o
        x2s = _dot(A_s, _dot(x1o, Ws2a) + _dot(x1s, Ws2b)) + b2s
        return jnp.concatenate([x2o, x2s], axis=1)

    x2 = gnn(xo_ref[:])
    x2a = gnn(xa_ref[:])

    rowmask = (lax.broadcasted_iota(jnp.int32, (NP, 1), 0) < N).astype(jnp.float32)
    h = jax.nn.sigmoid(jnp.sum(x2 * rowmask, axis=0, keepdims=True) / N)
    ha = jax.nn.sigmoid(jnp.sum(x2a * rowmask, axis=0, keepdims=True) / N)

    dW = dW_ref[:]
    db = db_ref[:]
    v = _dot_t(h, dW)
    va = _dot_t(ha, dW)
    sc1 = _dot_t(x2, v)
    sc2 = _dot_t(x2a, v)
    sc1a = _dot_t(x2a, va)
    sc2a = _dot_t(x2, va)
    ros = jnp.concatenate([sc1, sc2], axis=1) + db
    rosa = jnp.concatenate([sc1a, sc2a], axis=1) + db
    ros_ref[:] = ros[0:N, :]
    rosa_ref[:] = rosa[0:N, :]

    drug_k = _gip(x2[0:DRUG, :], DRUG)
    mic_k = _gip(x2[DRUG:N, :], MIC)
    out1 = _dot(drug_k, a1_ref[:])
    out2t = lax.dot_general(a2_ref[:], mic_k, (((0,), (0,)), ((), ())),
                            preferred_element_type=jnp.float32)
    out_ref[:] = (out1 + out2t) * 0.5
    x2_ref[:] = x2[0:N, :]


def _prep(x_o, x_a, edge_index_o, edge_index_s, W_o1, b_o1, W_s1, b_s1,
          W_o2, b_o2, W_s2, b_s2, disc_W, disc_b, alpha1, alpha2):
    xo_p = jnp.pad(x_o, ((0, NP - N), (0, 0)))
    xa_p = jnp.pad(x_a, ((0, NP - N), (0, 0)))
    eio_p = jnp.pad(edge_index_o, ((0, 0), (0, EP - E)), constant_values=NP - 1)
    eis_p = jnp.pad(edge_index_s, ((0, 0), (0, EP - E)), constant_values=NP - 1)
    return (xo_p, xa_p, eio_p, eis_p,
            W_o1, W_s1, W_o2, W_s2, disc_W,
            b_o1.reshape(1, H1), b_s1.reshape(1, H1),
            b_o2.reshape(1, H2), b_s2.reshape(1, H2),
            disc_b.reshape(1, 1), alpha1, alpha2)


@jax.jit
def _run(*args):
    return pl.pallas_call(_mngnn_kernel, out_shape=_OUT_SHAPES)(*args)


def kernel(x_o, x_a, edge_index_o, edge_index_s, W_o1, b_o1, W_s1, b_s1,
           W_o2, b_o2, W_s2, b_s2, disc_W, disc_b, alpha1, alpha2):
    out, ros, rosa, x2 = _run(*_prep(
        x_o, x_a, edge_index_o, edge_index_s, W_o1, b_o1, W_s1, b_s1,
        W_o2, b_o2, W_s2, b_s2, disc_W, disc_b, alpha1, alpha2))
    return (out, ros, rosa, x2)
